# baseline (speedup 1.0000x reference)
import functools

import jax
import jax.numpy as jnp
from jax import lax
from jax.experimental import pallas as pl
from jax.experimental.pallas import tpu as pltpu
from jax.experimental.pallas import tpu_sc as plsc

N = 10000
E = 160000
D_IN = 256
D_HID = 256
D_OUT = 40
EPS = 1e-5

NC = 2
NS = 16
CHUNK = 128

EP = ((E + NC * NS * CHUNK - 1) // (NC * NS * CHUNK)) * (NC * NS * CHUNK)
CH12 = EP // (NS * CHUNK)
CEDGE = 80
GR = 32
RING = 4
CHW = EP // (NC * NS * CHUNK)
NACC = N + NS
TROWS = 632
ZLAST = NACC - (NS - 1) * TROWS
OLAST = N - (NS - 1) * TROWS
BN = 1000
NB = N // BN

@functools.cache
def _mesh():
    return plsc.VectorSubcoreMesh(core_axis_name="c", subcore_axis_name="s",
                                  num_cores=NC, num_subcores=NS)


def _chunk_sizes(total, step):
    sizes = []
    left = total
    while left > 0:
        sizes.append(min(step, left))
        left -= step
    return tuple(sizes)


def _zero_acc(acc, zbuf, s, zrows):
    def fill(rows):
        base = s * TROWS
        off = 0
        for sz in _chunk_sizes(rows, zrows):
            pltpu.sync_copy(zbuf.at[pl.ds(0, sz)], acc.at[pl.ds(base + off, sz)])
            off += sz

    @pl.when(s < NS - 1)
    def _():
        fill(TROWS)

    @pl.when(s == NS - 1)
    def _():
        fill(ZLAST)


def _copy_out(acc, out_hbm, c, s):
    @pl.when(s < NS - 1)
    def _():
        pltpu.sync_copy(acc.at[pl.ds(s * TROWS, TROWS)],
                        out_hbm.at[pl.ds(c * N + s * TROWS, TROWS)])

    @pl.when(s == NS - 1)
    def _():
        pltpu.sync_copy(acc.at[pl.ds((NS - 1) * TROWS, OLAST)],
                        out_hbm.at[pl.ds(c * N + (NS - 1) * TROWS, OLAST)])


def _sc_deg(dst_r, ones128, zeros128):
    @functools.partial(
        pl.kernel,
        out_type=jax.ShapeDtypeStruct((2 * N, 128), jnp.float32),
        mesh=_mesh(),
        scratch_types=[
            pltpu.VMEM((CHW, CHUNK), jnp.int32),
            pltpu.VMEM((CHUNK, 128), jnp.float32),
            pltpu.VMEM((CHUNK, 128), jnp.float32),
            pltpu.VMEM_SHARED((NACC, 128), jnp.float32),
        ],
    )
    def deg_kernel(dst_hbm, ones_hbm, zeros_hbm, out_hbm, idx_v, ones_v, zer_v, acc):
        c = lax.axis_index("c")
        s = lax.axis_index("s")
        w = c * NS + s
        pltpu.sync_copy(zeros_hbm, zer_v)
        _zero_acc(acc, zer_v, s, CHUNK)
        pltpu.sync_copy(ones_hbm, ones_v)
        pltpu.sync_copy(dst_hbm.at[pl.ds(w * CHW, CHW)], idx_v)
        plsc.subcore_barrier()

        def body(j, carry):
            pltpu.sync_copy(ones_v, acc.at[idx_v.at[j]], add=True)
            return carry

        lax.fori_loop(0, CHW, body, 0)
        plsc.subcore_barrier()
        _copy_out(acc, out_hbm, c, s)

    return deg_kernel(dst_r, ones128, zeros128)


def _edge_pipe(u_hbm, src_hbm, dst_hbm, si_v, di_v, bufs, sems, acc,
               si_base, di_base, n_groups):
    for g in range(n_groups):
        pltpu.sync_copy(src_hbm.at[pl.ds(si_base + g * GR, GR)], si_v)
        pltpu.sync_copy(dst_hbm.at[pl.ds(di_base + g * GR, GR)], di_v)
        for b in range(RING - 1):
            pltpu.async_copy(u_hbm.at[si_v.at[b]], bufs[b], sems[b])

        def outer(q, carry):
            for b in range(RING):
                j = RING * q + b

                @pl.when(j + RING - 1 < GR)
                def _(j=j, b=b):
                    pltpu.async_copy(u_hbm.at[si_v.at[j + RING - 1]],
                                     bufs[(b + RING - 1) % RING],
                                     sems[(b + RING - 1) % RING])

                pltpu.make_async_copy(u_hbm.at[si_v.at[0]], bufs[b],
                                      sems[b]).wait()
                pltpu.sync_copy(bufs[b], acc.at[di_v.at[j]], add=True)
            return carry

        lax.fori_loop(0, GR // RING, outer, 0)


def _sc_agg128(u_flat, src2, dst_r, zeros128):
    @functools.partial(
        pl.kernel,
        out_type=jax.ShapeDtypeStruct((2 * N, 128), jnp.float32),
        mesh=_mesh(),
        scratch_types=[
            pltpu.VMEM((GR, CEDGE), jnp.int32),
            pltpu.VMEM((GR, CEDGE), jnp.int32),
            pltpu.VMEM((CEDGE, 128), jnp.float32),
            pltpu.VMEM((CEDGE, 128), jnp.float32),
            pltpu.VMEM((CEDGE, 128), jnp.float32),
            pltpu.VMEM((CEDGE, 128), jnp.float32),
            pltpu.VMEM_SHARED((NACC, 128), jnp.float32),
            pltpu.SemaphoreType.DMA,
            pltpu.SemaphoreType.DMA,
            pltpu.SemaphoreType.DMA,
            pltpu.SemaphoreType.DMA,
        ],
    )
    def agg_kernel(u_hbm, src_hbm, dst_hbm, zeros_hbm, out_hbm,
                   si_v, di_v, b0, b1, b2, b3, acc, s0, s1, s2, s3):
        c = lax.axis_index("c")
        s = lax.axis_index("s")
        pltpu.sync_copy(zeros_hbm, b0)
        _zero_acc(acc, b0, s, CEDGE)
        plsc.subcore_barrier()
        n_rows = EP // (NS * CEDGE)
        _edge_pipe(u_hbm, src_hbm, dst_hbm, si_v, di_v, (b0, b1, b2, b3),
                   (s0, s1, s2, s3), acc,
                   (c * NS + s) * n_rows, s * n_rows, n_rows // GR)
        plsc.subcore_barrier()
        _copy_out(acc, out_hbm, c, s)

    return agg_kernel(u_flat, src2, dst_r, zeros128)


def _sc_agg40(u3, src2, dst_r, zeros40):
    @functools.partial(
        pl.kernel,
        out_type=jax.ShapeDtypeStruct((2 * N, 128), jnp.float32),
        mesh=_mesh(),
        scratch_types=[
            pltpu.VMEM((GR, CEDGE), jnp.int32),
            pltpu.VMEM((GR, CEDGE), jnp.int32),
            pltpu.VMEM((CEDGE, 128), jnp.float32),
            pltpu.VMEM((CEDGE, 128), jnp.float32),
            pltpu.VMEM((CEDGE, 128), jnp.float32),
            pltpu.VMEM((CEDGE, 128), jnp.float32),
            pltpu.VMEM_SHARED((NACC, 128), jnp.float32),
            pltpu.SemaphoreType.DMA,
            pltpu.SemaphoreType.DMA,
            pltpu.SemaphoreType.DMA,
            pltpu.SemaphoreType.DMA,
        ],
    )
    def agg_kernel(u_hbm, src_hbm, dst_hbm, zeros_hbm, out_hbm,
                   si_v, di_v, b0, b1, b2, b3, acc, s0, s1, s2, s3):
        c = lax.axis_index("c")
        s = lax.axis_index("s")
        w = c * NS + s
        pltpu.sync_copy(zeros_hbm, b0)
        _zero_acc(acc, b0, s, CEDGE)
        plsc.subcore_barrier()
        n_rows = EP // (NC * NS * CEDGE)
        _edge_pipe(u_hbm, src_hbm, dst_hbm, si_v, di_v, (b0, b1, b2, b3),
                   (s0, s1, s2, s3), acc,
                   w * n_rows, w * n_rows, n_rows // GR)
        plsc.subcore_barrier()
        _copy_out(acc, out_hbm, c, s)

    return agg_kernel(u3, src2, dst_r, zeros40)


def _tc_dinv(deg_parts):
    def body(dp_ref, o_ref):
        deg = dp_ref[0, :] + dp_ref[1, :] + 1.0
        o_ref[...] = lax.rsqrt(deg)[:, None]

    return pl.pallas_call(
        body,
        out_shape=jax.ShapeDtypeStruct((N, 1), jnp.float32),
    )(deg_parts)


def _tc_mm1(x, w1r):
    def body(x_ref, w_ref, o_ref):
        o_ref[...] = lax.dot_general(x_ref[...].astype(jnp.bfloat16),
                                     w_ref[0].astype(jnp.bfloat16),
                                     (((1,), (1,)), ((), ())),
                                     preferred_element_type=jnp.float32)

    return pl.pallas_call(
        body,
        grid=(2, NB),
        in_specs=[
            pl.BlockSpec((BN, D_IN), lambda c, n: (n, 0)),
            pl.BlockSpec((1, 128, D_IN), lambda c, n: (c, 0, 0)),
        ],
        out_specs=pl.BlockSpec((BN, 128), lambda c, n: (c * NB + n, 0)),
        out_shape=jax.ShapeDtypeStruct((2 * N, 128), jnp.float32),
    )(x, w1r)


def _tc_scale(t, dinv):
    def body(t_ref, d_ref, o_ref):
        o_ref[...] = t_ref[...] * d_ref[...]

    return pl.pallas_call(
        body,
        grid=(2, NB),
        in_specs=[
            pl.BlockSpec((BN, 128), lambda c, n: (c * NB + n, 0)),
            pl.BlockSpec((BN, 1), lambda c, n: (n, 0)),
        ],
        out_specs=pl.BlockSpec((BN, 128), lambda c, n: (c * NB + n, 0)),
        out_shape=jax.ShapeDtypeStruct((2 * N, 128), jnp.float32),
    )(t, dinv)


def _tc_layer(agg, u, dinv, b_r, geff_r, beta_r, w_r):
    def body(a_ref, u_ref, d_ref, b_ref, g_ref, be_ref, w_ref, o_ref):
        d = d_ref[...][None]
        pre = (a_ref[...] + u_ref[...]) * d
        pre = pre + b_ref[...][:, None, :]
        h = jnp.maximum(pre * g_ref[...][:, None, :] + be_ref[...][:, None, :], 0.0)
        w = w_ref[0].astype(jnp.bfloat16)
        hb = h.astype(jnp.bfloat16)
        t = lax.dot_general(hb[0], w[:, :128], (((1,), (1,)), ((), ())),
                            preferred_element_type=jnp.float32)
        t = t + lax.dot_general(hb[1], w[:, 128:], (((1,), (1,)), ((), ())),
                                preferred_element_type=jnp.float32)
        o_ref[...] = t * d_ref[...]

    return pl.pallas_call(
        body,
        grid=(2, NB),
        in_specs=[
            pl.BlockSpec((2, BN, 128), lambda c, n: (0, n, 0)),
            pl.BlockSpec((2, BN, 128), lambda c, n: (0, n, 0)),
            pl.BlockSpec((BN, 1), lambda c, n: (n, 0)),
            pl.BlockSpec((2, 128), lambda c, n: (0, 0)),
            pl.BlockSpec((2, 128), lambda c, n: (0, 0)),
            pl.BlockSpec((2, 128), lambda c, n: (0, 0)),
            pl.BlockSpec((1, 128, D_HID), lambda c, n: (c, 0, 0)),
        ],
        out_specs=pl.BlockSpec((BN, 128), lambda c, n: (c * NB + n, 0)),
        out_shape=jax.ShapeDtypeStruct((2 * N, 128), jnp.float32),
    )(agg, u, dinv, b_r, geff_r, beta_r, w_r)


def _tc_layer3(agg, u, dinv, b_r, geff_r, beta_r, w3):
    def body(a_ref, u_ref, d_ref, b_ref, g_ref, be_ref, w_ref, o_ref):
        d = d_ref[...][None]
        pre = (a_ref[...] + u_ref[...]) * d
        pre = pre + b_ref[...][:, None, :]
        h = jnp.maximum(pre * g_ref[...][:, None, :] + be_ref[...][:, None, :], 0.0)
        w = w_ref[...].astype(jnp.bfloat16)
        hb = h.astype(jnp.bfloat16)
        t = lax.dot_general(hb[0], w[:, :128], (((1,), (1,)), ((), ())),
                            preferred_element_type=jnp.float32)
        t = t + lax.dot_general(hb[1], w[:, 128:], (((1,), (1,)), ((), ())),
                                preferred_element_type=jnp.float32)
        o_ref[...] = t * d_ref[...]

    return pl.pallas_call(
        body,
        grid=(NB,),
        in_specs=[
            pl.BlockSpec((2, BN, 128), lambda n: (0, n, 0)),
            pl.BlockSpec((2, BN, 128), lambda n: (0, n, 0)),
            pl.BlockSpec((BN, 1), lambda n: (n, 0)),
            pl.BlockSpec((2, 128), lambda n: (0, 0)),
            pl.BlockSpec((2, 128), lambda n: (0, 0)),
            pl.BlockSpec((2, 128), lambda n: (0, 0)),
            pl.BlockSpec((128, D_HID), lambda n: (0, 0)),
        ],
        out_specs=pl.BlockSpec((BN, 128), lambda n: (n, 0)),
        out_shape=jax.ShapeDtypeStruct((N, 128), jnp.float32),
    )(agg, u, dinv, b_r, geff_r, beta_r, w3)


def _tc_final(agg3, u3, dinv, b3):
    def body(a_ref, u_ref, d_ref, b_ref, o_ref):
        t = (a_ref[0] + a_ref[1] + u_ref[...]) * d_ref[...]
        o_ref[...] = t[:, :D_OUT] + b_ref[...]

    return pl.pallas_call(
        body,
        grid=(NB,),
        in_specs=[
            pl.BlockSpec((2, BN, 128), lambda n: (0, n, 0)),
            pl.BlockSpec((BN, 128), lambda n: (n, 0)),
            pl.BlockSpec((BN, 1), lambda n: (n, 0)),
            pl.BlockSpec((1, D_OUT), lambda n: (0, 0)),
        ],
        out_specs=pl.BlockSpec((BN, D_OUT), lambda n: (n, 0)),
        out_shape=jax.ShapeDtypeStruct((N, D_OUT), jnp.float32),
    )(agg3, u3, dinv, b3)


def kernel(x, edge_index, W1, b1, W2, b2, W3, b3, g1, beta1, g2, beta2):
    src = edge_index[0].astype(jnp.int32)
    dst = edge_index[1].astype(jnp.int32)
    npad = EP - E
    src_p = jnp.concatenate([src, jnp.zeros((npad,), jnp.int32)])
    dst_p = jnp.concatenate([dst, jnp.full((npad,), N, jnp.int32)])
    src2 = jnp.concatenate([src_p, src_p + N]).reshape(2 * EP // CEDGE, CEDGE)
    dst_r80 = dst_p.reshape(EP // CEDGE, CEDGE)
    dst_r = dst_p.reshape(EP // CHUNK, CHUNK)

    zeros128 = jnp.zeros((CHUNK, 128), jnp.float32)
    zeros80 = jnp.zeros((CEDGE, 128), jnp.float32)
    w3p = jnp.concatenate([W3, jnp.zeros((128 - D_OUT, D_HID), jnp.float32)])
    ones128 = jnp.ones((CHUNK, 128), jnp.float32)

    bn_scale = 1.0 / jnp.sqrt(1.0 + EPS)
    w1r = W1.reshape(2, 128, D_IN)
    w2r = W2.reshape(2, 128, D_HID)
    b1r = b1.reshape(2, 128)
    b2r = b2.reshape(2, 128)
    geff1 = (g1 * bn_scale).reshape(2, 128)
    geff2 = (g2 * bn_scale).reshape(2, 128)
    beta1r = beta1.reshape(2, 128)
    beta2r = beta2.reshape(2, 128)

    deg_parts = _sc_deg(dst_r, ones128, zeros128)
    t1 = _tc_mm1(x, w1r)
    dinv = _tc_dinv(deg_parts[:, 0].reshape(2, N))
    u1 = _tc_scale(t1, dinv)
    agg1 = _sc_agg128(u1, src2, dst_r80, zeros80)

    u2 = _tc_layer(agg1.reshape(2, N, 128), u1.reshape(2, N, 128), dinv,
                   b1r, geff1, beta1r, w2r)
    agg2 = _sc_agg128(u2, src2, dst_r80, zeros80)

    u3 = _tc_layer3(agg2.reshape(2, N, 128), u2.reshape(2, N, 128), dinv,
                    b2r, geff2, beta2r, w3p)
    agg3 = _sc_agg40(u3, src2, dst_r80, zeros80)

    return _tc_final(agg3.reshape(2, N, 128), u3, dinv, b3.reshape(1, D_OUT))


# speedup vs baseline: 2.2855x; 2.2855x over previous
import functools

import jax
import jax.numpy as jnp
from jax import lax
from jax.experimental import pallas as pl
from jax.experimental.pallas import tpu as pltpu
from jax.experimental.pallas import tpu_sc as plsc

N = 10000
E = 160000
D_IN = 256
D_HID = 256
D_OUT = 40
EPS = 1e-5

NC = 2
NS = 16
CHUNK = 128

EP = ((E + NC * NS * CHUNK - 1) // (NC * NS * CHUNK)) * (NC * NS * CHUNK)
CH12 = EP // (NS * CHUNK)
CEDGE = 80
GR = 32
RING = 4
CHW = EP // (NC * NS * CHUNK)
NACC = N + NS
TROWS = 632
ZLAST = NACC - (NS - 1) * TROWS
OLAST = N - (NS - 1) * TROWS
BN = 1000
NB = N // BN

@functools.cache
def _mesh():
    return plsc.VectorSubcoreMesh(core_axis_name="c", subcore_axis_name="s",
                                  num_cores=NC, num_subcores=NS)


def _chunk_sizes(total, step):
    sizes = []
    left = total
    while left > 0:
        sizes.append(min(step, left))
        left -= step
    return tuple(sizes)


def _zero_acc(acc, zbuf, s, zrows):
    def fill(rows):
        base = s * TROWS
        off = 0
        for sz in _chunk_sizes(rows, zrows):
            pltpu.sync_copy(zbuf.at[pl.ds(0, sz)], acc.at[pl.ds(base + off, sz)])
            off += sz

    @pl.when(s < NS - 1)
    def _():
        fill(TROWS)

    @pl.when(s == NS - 1)
    def _():
        fill(ZLAST)


def _copy_out(acc, out_hbm, c, s):
    @pl.when(s < NS - 1)
    def _():
        pltpu.sync_copy(acc.at[pl.ds(s * TROWS, TROWS)],
                        out_hbm.at[pl.ds(c * N + s * TROWS, TROWS)])

    @pl.when(s == NS - 1)
    def _():
        pltpu.sync_copy(acc.at[pl.ds((NS - 1) * TROWS, OLAST)],
                        out_hbm.at[pl.ds(c * N + (NS - 1) * TROWS, OLAST)])


def _sc_deg(dst_r, ones128, zeros128):
    @functools.partial(
        pl.kernel,
        out_type=jax.ShapeDtypeStruct((2 * N, 128), jnp.float32),
        mesh=_mesh(),
        scratch_types=[
            pltpu.VMEM((CHW, CHUNK), jnp.int32),
            pltpu.VMEM((CHUNK, 128), jnp.float32),
            pltpu.VMEM((CHUNK, 128), jnp.float32),
            pltpu.VMEM_SHARED((NACC, 128), jnp.float32),
        ],
    )
    def deg_kernel(dst_hbm, ones_hbm, zeros_hbm, out_hbm, idx_v, ones_v, zer_v, acc):
        c = lax.axis_index("c")
        s = lax.axis_index("s")
        w = c * NS + s
        pltpu.sync_copy(zeros_hbm, zer_v)
        _zero_acc(acc, zer_v, s, CHUNK)
        pltpu.sync_copy(ones_hbm, ones_v)
        pltpu.sync_copy(dst_hbm.at[pl.ds(w * CHW, CHW)], idx_v)
        plsc.subcore_barrier()

        def body(j, carry):
            pltpu.sync_copy(ones_v, acc.at[idx_v.at[j]], add=True)
            return carry

        lax.fori_loop(0, CHW, body, 0)
        plsc.subcore_barrier()
        _copy_out(acc, out_hbm, c, s)

    return deg_kernel(dst_r, ones128, zeros128)


def _edge_pipe(u_hbm, src_hbm, dst_hbm, si_v, di_v, bufs, sems, acc,
               si_base, di_base, n_groups):
    for g in range(n_groups):
        pltpu.sync_copy(src_hbm.at[pl.ds(si_base + g * GR, GR)], si_v)
        pltpu.sync_copy(dst_hbm.at[pl.ds(di_base + g * GR, GR)], di_v)
        for b in range(RING - 1):
            pltpu.async_copy(u_hbm.at[si_v.at[b]], bufs[b], sems[b])

        def outer(q, carry):
            for b in range(RING):
                j = RING * q + b

                @pl.when(j + RING - 1 < GR)
                def _(j=j, b=b):
                    pltpu.async_copy(u_hbm.at[si_v.at[j + RING - 1]],
                                     bufs[(b + RING - 1) % RING],
                                     sems[(b + RING - 1) % RING])

                pltpu.make_async_copy(u_hbm.at[si_v.at[0]], bufs[b],
                                      sems[b]).wait()
                pltpu.sync_copy(bufs[b], acc.at[di_v.at[j]], add=True)
            return carry

        lax.fori_loop(0, GR // RING, outer, 0)


def _sc_agg128(u_flat, src2, dst_r, zeros128):
    @functools.partial(
        pl.kernel,
        out_type=jax.ShapeDtypeStruct((2 * N, 128), jnp.float32),
        mesh=_mesh(),
        scratch_types=[
            pltpu.VMEM((GR, CEDGE), jnp.int32),
            pltpu.VMEM((GR, CEDGE), jnp.int32),
            pltpu.VMEM((CEDGE, 128), jnp.float32),
            pltpu.VMEM((CEDGE, 128), jnp.float32),
            pltpu.VMEM((CEDGE, 128), jnp.float32),
            pltpu.VMEM((CEDGE, 128), jnp.float32),
            pltpu.VMEM_SHARED((NACC, 128), jnp.float32),
            pltpu.SemaphoreType.DMA,
            pltpu.SemaphoreType.DMA,
            pltpu.SemaphoreType.DMA,
            pltpu.SemaphoreType.DMA,
        ],
    )
    def agg_kernel(u_hbm, src_hbm, dst_hbm, zeros_hbm, out_hbm,
                   si_v, di_v, b0, b1, b2, b3, acc, s0, s1, s2, s3):
        c = lax.axis_index("c")
        s = lax.axis_index("s")
        pltpu.sync_copy(zeros_hbm, b0)
        _zero_acc(acc, b0, s, CEDGE)
        plsc.subcore_barrier()
        n_rows = EP // (NS * CEDGE)
        _edge_pipe(u_hbm, src_hbm, dst_hbm, si_v, di_v, (b0, b1, b2, b3),
                   (s0, s1, s2, s3), acc,
                   (c * NS + s) * n_rows, s * n_rows, n_rows // GR)
        plsc.subcore_barrier()
        _copy_out(acc, out_hbm, c, s)

    return agg_kernel(u_flat, src2, dst_r, zeros128)


def _sc_agg40(u3, src2, dst_r, zeros40):
    @functools.partial(
        pl.kernel,
        out_type=jax.ShapeDtypeStruct((2 * N, 128), jnp.float32),
        mesh=_mesh(),
        scratch_types=[
            pltpu.VMEM((GR, CEDGE), jnp.int32),
            pltpu.VMEM((GR, CEDGE), jnp.int32),
            pltpu.VMEM((CEDGE, 128), jnp.float32),
            pltpu.VMEM((CEDGE, 128), jnp.float32),
            pltpu.VMEM((CEDGE, 128), jnp.float32),
            pltpu.VMEM((CEDGE, 128), jnp.float32),
            pltpu.VMEM_SHARED((NACC, 128), jnp.float32),
            pltpu.SemaphoreType.DMA,
            pltpu.SemaphoreType.DMA,
            pltpu.SemaphoreType.DMA,
            pltpu.SemaphoreType.DMA,
        ],
    )
    def agg_kernel(u_hbm, src_hbm, dst_hbm, zeros_hbm, out_hbm,
                   si_v, di_v, b0, b1, b2, b3, acc, s0, s1, s2, s3):
        c = lax.axis_index("c")
        s = lax.axis_index("s")
        w = c * NS + s
        pltpu.sync_copy(zeros_hbm, b0)
        _zero_acc(acc, b0, s, CEDGE)
        plsc.subcore_barrier()
        n_rows = EP // (NC * NS * CEDGE)
        _edge_pipe(u_hbm, src_hbm, dst_hbm, si_v, di_v, (b0, b1, b2, b3),
                   (s0, s1, s2, s3), acc,
                   w * n_rows, w * n_rows, n_rows // GR)
        plsc.subcore_barrier()
        _copy_out(acc, out_hbm, c, s)

    return agg_kernel(u3, src2, dst_r, zeros40)


def _tc_dinv(deg_parts):
    def body(dp_ref, o_ref):
        deg = dp_ref[0, :] + dp_ref[1, :] + 1.0
        o_ref[...] = lax.rsqrt(deg)[:, None]

    return pl.pallas_call(
        body,
        out_shape=jax.ShapeDtypeStruct((N, 1), jnp.float32),
    )(deg_parts)


def _tc_mm1(x, w1r):
    def body(x_ref, w_ref, o_ref):
        o_ref[...] = lax.dot_general(x_ref[...], w_ref[0],
                                     (((1,), (1,)), ((), ())),
                                     preferred_element_type=jnp.float32)

    return pl.pallas_call(
        body,
        grid=(2, NB),
        in_specs=[
            pl.BlockSpec((BN, D_IN), lambda c, n: (n, 0)),
            pl.BlockSpec((1, 128, D_IN), lambda c, n: (c, 0, 0)),
        ],
        out_specs=pl.BlockSpec((BN, 128), lambda c, n: (c * NB + n, 0)),
        out_shape=jax.ShapeDtypeStruct((2 * N, 128), jnp.float32),
    )(x, w1r)


def _tc_scale(t, dinv):
    def body(t_ref, d_ref, o_ref):
        o_ref[...] = t_ref[...] * d_ref[...]

    return pl.pallas_call(
        body,
        grid=(2, NB),
        in_specs=[
            pl.BlockSpec((BN, 128), lambda c, n: (c * NB + n, 0)),
            pl.BlockSpec((BN, 1), lambda c, n: (n, 0)),
        ],
        out_specs=pl.BlockSpec((BN, 128), lambda c, n: (c * NB + n, 0)),
        out_shape=jax.ShapeDtypeStruct((2 * N, 128), jnp.float32),
    )(t, dinv)


def _tc_layer(agg, u, dinv, b_r, geff_r, beta_r, w_r):
    def body(a_ref, u_ref, d_ref, b_ref, g_ref, be_ref, w_ref, o_ref):
        d = d_ref[...][None]
        pre = (a_ref[...] + u_ref[...]) * d
        pre = pre + b_ref[...][:, None, :]
        h = jnp.maximum(pre * g_ref[...][:, None, :] + be_ref[...][:, None, :], 0.0)
        w = w_ref[0]
        t = lax.dot_general(h[0], w[:, :128], (((1,), (1,)), ((), ())),
                            preferred_element_type=jnp.float32)
        t = t + lax.dot_general(h[1], w[:, 128:], (((1,), (1,)), ((), ())),
                                preferred_element_type=jnp.float32)
        o_ref[...] = t * d_ref[...]

    return pl.pallas_call(
        body,
        grid=(2, NB),
        in_specs=[
            pl.BlockSpec((2, BN, 128), lambda c, n: (0, n, 0)),
            pl.BlockSpec((2, BN, 128), lambda c, n: (0, n, 0)),
            pl.BlockSpec((BN, 1), lambda c, n: (n, 0)),
            pl.BlockSpec((2, 128), lambda c, n: (0, 0)),
            pl.BlockSpec((2, 128), lambda c, n: (0, 0)),
            pl.BlockSpec((2, 128), lambda c, n: (0, 0)),
            pl.BlockSpec((1, 128, D_HID), lambda c, n: (c, 0, 0)),
        ],
        out_specs=pl.BlockSpec((BN, 128), lambda c, n: (c * NB + n, 0)),
        out_shape=jax.ShapeDtypeStruct((2 * N, 128), jnp.float32),
    )(agg, u, dinv, b_r, geff_r, beta_r, w_r)


def _tc_layer3(agg, u, dinv, b_r, geff_r, beta_r, w3):
    def body(a_ref, u_ref, d_ref, b_ref, g_ref, be_ref, w_ref, o_ref):
        d = d_ref[...][None]
        pre = (a_ref[...] + u_ref[...]) * d
        pre = pre + b_ref[...][:, None, :]
        h = jnp.maximum(pre * g_ref[...][:, None, :] + be_ref[...][:, None, :], 0.0)
        w = w_ref[...]
        t = lax.dot_general(h[0], w[:, :128], (((1,), (1,)), ((), ())),
                            preferred_element_type=jnp.float32)
        t = t + lax.dot_general(h[1], w[:, 128:], (((1,), (1,)), ((), ())),
                                preferred_element_type=jnp.float32)
        o_ref[...] = t * d_ref[...]

    return pl.pallas_call(
        body,
        grid=(NB,),
        in_specs=[
            pl.BlockSpec((2, BN, 128), lambda n: (0, n, 0)),
            pl.BlockSpec((2, BN, 128), lambda n: (0, n, 0)),
            pl.BlockSpec((BN, 1), lambda n: (n, 0)),
            pl.BlockSpec((2, 128), lambda n: (0, 0)),
            pl.BlockSpec((2, 128), lambda n: (0, 0)),
            pl.BlockSpec((2, 128), lambda n: (0, 0)),
            pl.BlockSpec((128, D_HID), lambda n: (0, 0)),
        ],
        out_specs=pl.BlockSpec((BN, 128), lambda n: (n, 0)),
        out_shape=jax.ShapeDtypeStruct((N, 128), jnp.float32),
    )(agg, u, dinv, b_r, geff_r, beta_r, w3)


def _tc_final(agg3, u3, dinv, b3):
    def body(a_ref, u_ref, d_ref, b_ref, o_ref):
        t = (a_ref[0] + a_ref[1] + u_ref[...]) * d_ref[...]
        o_ref[...] = t[:, :D_OUT] + b_ref[...]

    return pl.pallas_call(
        body,
        grid=(NB,),
        in_specs=[
            pl.BlockSpec((2, BN, 128), lambda n: (0, n, 0)),
            pl.BlockSpec((BN, 128), lambda n: (n, 0)),
            pl.BlockSpec((BN, 1), lambda n: (n, 0)),
            pl.BlockSpec((1, D_OUT), lambda n: (0, 0)),
        ],
        out_specs=pl.BlockSpec((BN, D_OUT), lambda n: (n, 0)),
        out_shape=jax.ShapeDtypeStruct((N, D_OUT), jnp.float32),
    )(agg3, u3, dinv, b3)


def kernel(x, edge_index, W1, b1, W2, b2, W3, b3, g1, beta1, g2, beta2):
    src = edge_index[0].astype(jnp.int32)
    dst = edge_index[1].astype(jnp.int32)
    npad = EP - E
    pad_ar = jnp.arange(npad, dtype=jnp.int32)
    src_p = jnp.concatenate([src, pad_ar % N])
    dst_p = jnp.concatenate([dst, N + (pad_ar % NS)])
    src2 = jnp.concatenate([src_p, src_p + N]).reshape(2 * EP // CEDGE, CEDGE)
    dst_r80 = dst_p.reshape(EP // CEDGE, CEDGE)
    dst_r = dst_p.reshape(EP // CHUNK, CHUNK)

    zeros128 = jnp.zeros((CHUNK, 128), jnp.float32)
    zeros80 = jnp.zeros((CEDGE, 128), jnp.float32)
    w3p = jnp.concatenate([W3, jnp.zeros((128 - D_OUT, D_HID), jnp.float32)])
    ones128 = jnp.ones((CHUNK, 128), jnp.float32)

    bn_scale = 1.0 / jnp.sqrt(1.0 + EPS)
    w1r = W1.reshape(2, 128, D_IN)
    w2r = W2.reshape(2, 128, D_HID)
    b1r = b1.reshape(2, 128)
    b2r = b2.reshape(2, 128)
    geff1 = (g1 * bn_scale).reshape(2, 128)
    geff2 = (g2 * bn_scale).reshape(2, 128)
    beta1r = beta1.reshape(2, 128)
    beta2r = beta2.reshape(2, 128)

    deg_parts = _sc_deg(dst_r, ones128, zeros128)
    t1 = _tc_mm1(x, w1r)
    dinv = _tc_dinv(deg_parts[:, 0].reshape(2, N))
    u1 = _tc_scale(t1, dinv)
    agg1 = _sc_agg128(u1, src2, dst_r80, zeros80)

    u2 = _tc_layer(agg1.reshape(2, N, 128), u1.reshape(2, N, 128), dinv,
                   b1r, geff1, beta1r, w2r)
    agg2 = _sc_agg128(u2, src2, dst_r80, zeros80)

    u3 = _tc_layer3(agg2.reshape(2, N, 128), u2.reshape(2, N, 128), dinv,
                    b2r, geff2, beta2r, w3p)
    agg3 = _sc_agg40(u3, src2, dst_r80, zeros80)

    return _tc_final(agg3.reshape(2, N, 128), u3, dinv, b3.reshape(1, D_OUT))


# speedup vs baseline: 2.3479x; 1.0273x over previous
import functools

import jax
import jax.numpy as jnp
from jax import lax
from jax.experimental import pallas as pl
from jax.experimental.pallas import tpu as pltpu
from jax.experimental.pallas import tpu_sc as plsc

N = 10000
E = 160000
D_IN = 256
D_HID = 256
D_OUT = 40
EPS = 1e-5

NC = 2
NS = 16
CHUNK = 128

EP = ((E + NC * NS * CHUNK - 1) // (NC * NS * CHUNK)) * (NC * NS * CHUNK)
CH12 = EP // (NS * CHUNK)
CEDGE = 80
GR = 32
RING = 4
CHW = EP // (NC * NS * CHUNK)
NACC = N + NS
TROWS = 632
ZLAST = NACC - (NS - 1) * TROWS
OLAST = N - (NS - 1) * TROWS
BN = 1000
NB = N // BN

@functools.cache
def _mesh():
    return plsc.VectorSubcoreMesh(core_axis_name="c", subcore_axis_name="s",
                                  num_cores=NC, num_subcores=NS)


def _chunk_sizes(total, step):
    sizes = []
    left = total
    while left > 0:
        sizes.append(min(step, left))
        left -= step
    return tuple(sizes)


def _zero_acc(acc, zbuf, s, zrows):
    def fill(rows):
        base = s * TROWS
        off = 0
        for sz in _chunk_sizes(rows, zrows):
            pltpu.sync_copy(zbuf.at[pl.ds(0, sz)], acc.at[pl.ds(base + off, sz)])
            off += sz

    @pl.when(s < NS - 1)
    def _():
        fill(TROWS)

    @pl.when(s == NS - 1)
    def _():
        fill(ZLAST)


def _copy_out(acc, out_hbm, c, s):
    @pl.when(s < NS - 1)
    def _():
        pltpu.sync_copy(acc.at[pl.ds(s * TROWS, TROWS)],
                        out_hbm.at[pl.ds(c * N + s * TROWS, TROWS)])

    @pl.when(s == NS - 1)
    def _():
        pltpu.sync_copy(acc.at[pl.ds((NS - 1) * TROWS, OLAST)],
                        out_hbm.at[pl.ds(c * N + (NS - 1) * TROWS, OLAST)])


def _sc_deg(dst_r, ones128, zeros128):
    @functools.partial(
        pl.kernel,
        out_type=jax.ShapeDtypeStruct((2 * N, 128), jnp.float32),
        mesh=_mesh(),
        scratch_types=[
            pltpu.VMEM((CHW, CHUNK), jnp.int32),
            pltpu.VMEM((CHUNK, 128), jnp.float32),
            pltpu.VMEM((CHUNK, 128), jnp.float32),
            pltpu.VMEM_SHARED((NACC, 128), jnp.float32),
        ],
    )
    def deg_kernel(dst_hbm, ones_hbm, zeros_hbm, out_hbm, idx_v, ones_v, zer_v, acc):
        c = lax.axis_index("c")
        s = lax.axis_index("s")
        w = c * NS + s
        pltpu.sync_copy(zeros_hbm, zer_v)
        _zero_acc(acc, zer_v, s, CHUNK)
        pltpu.sync_copy(ones_hbm, ones_v)
        pltpu.sync_copy(dst_hbm.at[pl.ds(w * CHW, CHW)], idx_v)
        plsc.subcore_barrier()

        def body(j, carry):
            pltpu.sync_copy(ones_v, acc.at[idx_v.at[j]], add=True)
            return carry

        lax.fori_loop(0, CHW, body, 0)
        plsc.subcore_barrier()
        _copy_out(acc, out_hbm, c, s)

    return deg_kernel(dst_r, ones128, zeros128)


def _edge_pipe(u_hbm, src_hbm, dst_hbm, si_v, di_v, bufs, sems, acc,
               si_base, di_base, n_groups):
    for g in range(n_groups):
        pltpu.sync_copy(src_hbm.at[pl.ds(si_base + g * GR, GR)], si_v)
        pltpu.sync_copy(dst_hbm.at[pl.ds(di_base + g * GR, GR)], di_v)
        for b in range(RING - 1):
            pltpu.async_copy(u_hbm.at[si_v.at[b]], bufs[b], sems[b])

        def outer(q, carry):
            for b in range(RING):
                j = RING * q + b

                @pl.when(j + RING - 1 < GR)
                def _(j=j, b=b):
                    pltpu.async_copy(u_hbm.at[si_v.at[j + RING - 1]],
                                     bufs[(b + RING - 1) % RING],
                                     sems[(b + RING - 1) % RING])

                pltpu.make_async_copy(u_hbm.at[si_v.at[0]], bufs[b],
                                      sems[b]).wait()
                pltpu.sync_copy(bufs[b], acc.at[di_v.at[j]], add=True)
            return carry

        lax.fori_loop(0, GR // RING, outer, 0)


def _sc_agg128(u_flat, src2, dst_r, zeros128):
    @functools.partial(
        pl.kernel,
        out_type=jax.ShapeDtypeStruct((2 * N, 128), jnp.float32),
        mesh=_mesh(),
        scratch_types=[
            pltpu.VMEM((GR, CEDGE), jnp.int32),
            pltpu.VMEM((GR, CEDGE), jnp.int32),
            pltpu.VMEM((CEDGE, 128), jnp.float32),
            pltpu.VMEM((CEDGE, 128), jnp.float32),
            pltpu.VMEM((CEDGE, 128), jnp.float32),
            pltpu.VMEM((CEDGE, 128), jnp.float32),
            pltpu.VMEM_SHARED((NACC, 128), jnp.float32),
            pltpu.SemaphoreType.DMA,
            pltpu.SemaphoreType.DMA,
            pltpu.SemaphoreType.DMA,
            pltpu.SemaphoreType.DMA,
        ],
    )
    def agg_kernel(u_hbm, src_hbm, dst_hbm, zeros_hbm, out_hbm,
                   si_v, di_v, b0, b1, b2, b3, acc, s0, s1, s2, s3):
        c = lax.axis_index("c")
        s = lax.axis_index("s")
        pltpu.sync_copy(zeros_hbm, b0)
        _zero_acc(acc, b0, s, CEDGE)
        plsc.subcore_barrier()
        n_rows = EP // (NS * CEDGE)
        _edge_pipe(u_hbm, src_hbm, dst_hbm, si_v, di_v, (b0, b1, b2, b3),
                   (s0, s1, s2, s3), acc,
                   (c * NS + s) * n_rows, s * n_rows, n_rows // GR)
        plsc.subcore_barrier()
        _copy_out(acc, out_hbm, c, s)

    return agg_kernel(u_flat, src2, dst_r, zeros128)


def _sc_agg40(u3, src2, dst_r, zeros40):
    @functools.partial(
        pl.kernel,
        out_type=jax.ShapeDtypeStruct((2 * N, 128), jnp.float32),
        mesh=_mesh(),
        scratch_types=[
            pltpu.VMEM((GR, CEDGE), jnp.int32),
            pltpu.VMEM((GR, CEDGE), jnp.int32),
            pltpu.VMEM((CEDGE, 128), jnp.float32),
            pltpu.VMEM((CEDGE, 128), jnp.float32),
            pltpu.VMEM((CEDGE, 128), jnp.float32),
            pltpu.VMEM((CEDGE, 128), jnp.float32),
            pltpu.VMEM_SHARED((NACC, 128), jnp.float32),
            pltpu.SemaphoreType.DMA,
            pltpu.SemaphoreType.DMA,
            pltpu.SemaphoreType.DMA,
            pltpu.SemaphoreType.DMA,
        ],
    )
    def agg_kernel(u_hbm, src_hbm, dst_hbm, zeros_hbm, out_hbm,
                   si_v, di_v, b0, b1, b2, b3, acc, s0, s1, s2, s3):
        c = lax.axis_index("c")
        s = lax.axis_index("s")
        w = c * NS + s
        pltpu.sync_copy(zeros_hbm, b0)
        _zero_acc(acc, b0, s, CEDGE)
        plsc.subcore_barrier()
        n_rows = EP // (NC * NS * CEDGE)
        _edge_pipe(u_hbm, src_hbm, dst_hbm, si_v, di_v, (b0, b1, b2, b3),
                   (s0, s1, s2, s3), acc,
                   w * n_rows, w * n_rows, n_rows // GR)
        plsc.subcore_barrier()
        _copy_out(acc, out_hbm, c, s)

    return agg_kernel(u3, src2, dst_r, zeros40)


def _tc_dinv(deg_parts):
    def body(dp_ref, o_ref):
        deg = dp_ref[0, :] + dp_ref[1, :] + 1.0
        o_ref[...] = lax.rsqrt(deg)[:, None]

    return pl.pallas_call(
        body,
        out_shape=jax.ShapeDtypeStruct((N, 1), jnp.float32),
    )(deg_parts)


def _tc_mm1(x, w1r):
    def body(x_ref, w_ref, o_ref):
        o_ref[...] = lax.dot_general(x_ref[...].astype(jnp.bfloat16),
                                     w_ref[0].astype(jnp.bfloat16),
                                     (((1,), (1,)), ((), ())),
                                     preferred_element_type=jnp.float32)

    return pl.pallas_call(
        body,
        grid=(2, NB),
        in_specs=[
            pl.BlockSpec((BN, D_IN), lambda c, n: (n, 0)),
            pl.BlockSpec((1, 128, D_IN), lambda c, n: (c, 0, 0)),
        ],
        out_specs=pl.BlockSpec((BN, 128), lambda c, n: (c * NB + n, 0)),
        out_shape=jax.ShapeDtypeStruct((2 * N, 128), jnp.float32),
    )(x, w1r)


def _tc_scale(t, dinv):
    def body(t_ref, d_ref, o_ref):
        o_ref[...] = t_ref[...] * d_ref[...]

    return pl.pallas_call(
        body,
        grid=(2, NB),
        in_specs=[
            pl.BlockSpec((BN, 128), lambda c, n: (c * NB + n, 0)),
            pl.BlockSpec((BN, 1), lambda c, n: (n, 0)),
        ],
        out_specs=pl.BlockSpec((BN, 128), lambda c, n: (c * NB + n, 0)),
        out_shape=jax.ShapeDtypeStruct((2 * N, 128), jnp.float32),
    )(t, dinv)


def _tc_layer(agg, u, dinv, b_r, geff_r, beta_r, w_r):
    def body(a_ref, u_ref, d_ref, b_ref, g_ref, be_ref, w_ref, o_ref):
        d = d_ref[...][None]
        pre = (a_ref[...] + u_ref[...]) * d
        pre = pre + b_ref[...][:, None, :]
        h = jnp.maximum(pre * g_ref[...][:, None, :] + be_ref[...][:, None, :], 0.0)
        hb = jnp.concatenate([h[0], h[1]], axis=1).astype(jnp.bfloat16)
        wb = w_ref[...].astype(jnp.bfloat16)
        t0 = lax.dot_general(hb, wb[0], (((1,), (1,)), ((), ())),
                             preferred_element_type=jnp.float32)
        t1 = lax.dot_general(hb, wb[1], (((1,), (1,)), ((), ())),
                             preferred_element_type=jnp.float32)
        o_ref[...] = jnp.stack([t0, t1]) * d

    return pl.pallas_call(
        body,
        grid=(NB,),
        in_specs=[
            pl.BlockSpec((2, BN, 128), lambda n: (0, n, 0)),
            pl.BlockSpec((2, BN, 128), lambda n: (0, n, 0)),
            pl.BlockSpec((BN, 1), lambda n: (n, 0)),
            pl.BlockSpec((2, 128), lambda n: (0, 0)),
            pl.BlockSpec((2, 128), lambda n: (0, 0)),
            pl.BlockSpec((2, 128), lambda n: (0, 0)),
            pl.BlockSpec((2, 128, D_HID), lambda n: (0, 0, 0)),
        ],
        out_specs=pl.BlockSpec((2, BN, 128), lambda n: (0, n, 0)),
        out_shape=jax.ShapeDtypeStruct((2, N, 128), jnp.float32),
    )(agg, u, dinv, b_r, geff_r, beta_r, w_r)


def _tc_layer3(agg, u, dinv, b_r, geff_r, beta_r, w3):
    def body(a_ref, u_ref, d_ref, b_ref, g_ref, be_ref, w_ref, o_ref):
        d = d_ref[...][None]
        pre = (a_ref[...] + u_ref[...]) * d
        pre = pre + b_ref[...][:, None, :]
        h = jnp.maximum(pre * g_ref[...][:, None, :] + be_ref[...][:, None, :], 0.0)
        hb = jnp.concatenate([h[0], h[1]], axis=1).astype(jnp.bfloat16)
        wb = w_ref[...].astype(jnp.bfloat16)
        t = lax.dot_general(hb, wb, (((1,), (1,)), ((), ())),
                            preferred_element_type=jnp.float32)
        o_ref[...] = t * d_ref[...]

    return pl.pallas_call(
        body,
        grid=(NB,),
        in_specs=[
            pl.BlockSpec((2, BN, 128), lambda n: (0, n, 0)),
            pl.BlockSpec((2, BN, 128), lambda n: (0, n, 0)),
            pl.BlockSpec((BN, 1), lambda n: (n, 0)),
            pl.BlockSpec((2, 128), lambda n: (0, 0)),
            pl.BlockSpec((2, 128), lambda n: (0, 0)),
            pl.BlockSpec((2, 128), lambda n: (0, 0)),
            pl.BlockSpec((128, D_HID), lambda n: (0, 0)),
        ],
        out_specs=pl.BlockSpec((BN, 128), lambda n: (n, 0)),
        out_shape=jax.ShapeDtypeStruct((N, 128), jnp.float32),
    )(agg, u, dinv, b_r, geff_r, beta_r, w3)


def _tc_final(agg3, u3, dinv, b3):
    def body(a_ref, u_ref, d_ref, b_ref, o_ref):
        t = (a_ref[0] + a_ref[1] + u_ref[...]) * d_ref[...]
        o_ref[...] = t[:, :D_OUT] + b_ref[...]

    return pl.pallas_call(
        body,
        grid=(NB,),
        in_specs=[
            pl.BlockSpec((2, BN, 128), lambda n: (0, n, 0)),
            pl.BlockSpec((BN, 128), lambda n: (n, 0)),
            pl.BlockSpec((BN, 1), lambda n: (n, 0)),
            pl.BlockSpec((1, D_OUT), lambda n: (0, 0)),
        ],
        out_specs=pl.BlockSpec((BN, D_OUT), lambda n: (n, 0)),
        out_shape=jax.ShapeDtypeStruct((N, D_OUT), jnp.float32),
    )(agg3, u3, dinv, b3)


def kernel(x, edge_index, W1, b1, W2, b2, W3, b3, g1, beta1, g2, beta2):
    src = edge_index[0].astype(jnp.int32)
    dst = edge_index[1].astype(jnp.int32)
    npad = EP - E
    pad_ar = jnp.arange(npad, dtype=jnp.int32)
    src_p = jnp.concatenate([src, pad_ar % N])
    dst_p = jnp.concatenate([dst, N + (pad_ar % NS)])
    src2 = jnp.concatenate([src_p, src_p + N]).reshape(2 * EP // CEDGE, CEDGE)
    dst_r80 = dst_p.reshape(EP // CEDGE, CEDGE)
    dst_r = dst_p.reshape(EP // CHUNK, CHUNK)

    zeros128 = jnp.zeros((CHUNK, 128), jnp.float32)
    zeros80 = jnp.zeros((CEDGE, 128), jnp.float32)
    w3p = jnp.concatenate([W3, jnp.zeros((128 - D_OUT, D_HID), jnp.float32)])
    ones128 = jnp.ones((CHUNK, 128), jnp.float32)

    bn_scale = 1.0 / jnp.sqrt(1.0 + EPS)
    w1r = W1.reshape(2, 128, D_IN)
    w2r = W2.reshape(2, 128, D_HID)
    b1r = b1.reshape(2, 128)
    b2r = b2.reshape(2, 128)
    geff1 = (g1 * bn_scale).reshape(2, 128)
    geff2 = (g2 * bn_scale).reshape(2, 128)
    beta1r = beta1.reshape(2, 128)
    beta2r = beta2.reshape(2, 128)

    deg_parts = _sc_deg(dst_r, ones128, zeros128)
    t1 = _tc_mm1(x, w1r)
    dinv = _tc_dinv(deg_parts[:, 0].reshape(2, N))
    u1 = _tc_scale(t1, dinv)
    agg1 = _sc_agg128(u1, src2, dst_r80, zeros80)

    u2 = _tc_layer(agg1.reshape(2, N, 128), u1.reshape(2, N, 128), dinv,
                   b1r, geff1, beta1r, w2r)
    agg2 = _sc_agg128(u2.reshape(2 * N, 128), src2, dst_r80, zeros80)

    u3 = _tc_layer3(agg2.reshape(2, N, 128), u2, dinv,
                    b2r, geff2, beta2r, w3p)
    agg3 = _sc_agg40(u3, src2, dst_r80, zeros80)

    return _tc_final(agg3.reshape(2, N, 128), u3, dinv, b3.reshape(1, D_OUT))
